# Initial kernel scaffold; baseline (speedup 1.0000x reference)
import jax
import jax.numpy as jnp
from jax.experimental import pallas as pl


def kernel(kp1, w_kp1, kp1_desc, desc2, homo12):
    raise NotImplementedError("write your pallas kernel here")


# speedup vs baseline: 918.5786x; 918.5786x over previous
import functools

import jax
import jax.numpy as jnp
from jax.experimental import pallas as pl
from jax.experimental.pallas import tpu as pltpu

_GRID = 16.0
_NUM_NEG = 16
_SOS_NEG = 8
_MARGIN = 1.0


def _nearest4(x, y):
    n = x.shape[0]
    jx = jnp.clip(jnp.floor(x * (1.0 / _GRID)), 0.0, 31.0)
    jy = jnp.clip(jnp.floor(y * (1.0 / _GRID)), 0.0, 31.0)
    c0 = jnp.clip(jx - 2.0, 0.0, 27.0)
    r0 = jnp.clip(jy - 2.0, 0.0, 27.0)
    lane = jax.lax.broadcasted_iota(jnp.int32, (n, 25), 1).astype(jnp.float32)
    dcol = lane - 5.0 * jnp.floor(lane * 0.2)
    drow = jnp.floor(lane * 0.2)
    cols = c0 + dcol
    rows = r0 + drow
    cx = cols * _GRID + 8.0
    cy = rows * _GRID + 8.0
    dx = x - cx
    dy = y - cy
    d2 = dx * dx + dy * dy
    idx = rows * 32.0 + cols
    big = jnp.float32(1e30)
    ids = []
    for _ in range(4):
        m = jnp.min(d2, axis=1, keepdims=True)
        sel = jnp.min(jnp.where(d2 == m, idx, jnp.float32(4096.0)),
                      axis=1, keepdims=True)
        ids.append(sel)
        d2 = jnp.where(idx == sel, big, d2)
    return ids


def _counts(ids, lane_hw):
    acc = None
    for s in ids:
        plane = (lane_hw == s).astype(jnp.float32)
        acc = plane if acc is None else acc + plane
    return acc


def _loss_kernel(homo_ref, kp1_ref, wkp1_ref, kd_ref, d_ref, dt_ref, out_ref):
    i = pl.program_id(0)
    n = kd_ref.shape[1]
    hw = d_ref.shape[2]

    kd = kd_ref[0]
    D = d_ref[0]
    Dt = dt_ref[0]

    kx = kp1_ref[0, :, 0:1]
    ky = kp1_ref[0, :, 1:2]
    wx = wkp1_ref[0, :, 0:1]
    wy = wkp1_ref[0, :, 1:2]

    lane_hw = jax.lax.broadcasted_iota(jnp.int32, (n, hw), 1).astype(jnp.float32)

    kids = _nearest4(kx, ky)
    wids = _nearest4(wx, wy)
    K = _counts(kids, lane_hw)
    W = _counts(wids, lane_hw)

    h00 = homo_ref[i, 0]
    h01 = homo_ref[i, 1]
    h02 = homo_ref[i, 2]
    h10 = homo_ref[i, 3]
    h11 = homo_ref[i, 4]
    h12 = homo_ref[i, 5]
    h20 = homo_ref[i, 6]
    h21 = homo_ref[i, 7]
    h22 = homo_ref[i, 8]
    N = None
    for p in range(4):
        cid = kids[p]
        col = cid - 32.0 * jnp.floor(cid * (1.0 / 32.0))
        row = jnp.floor(cid * (1.0 / 32.0))
        cx = col * _GRID + 8.0
        cy = row * _GRID + 8.0
        wz = h20 * cx + h21 * cy + h22
        px = (h00 * cx + h01 * cy + h02) / (wz + 1e-8)
        py = (h10 * cx + h11 * cy + h12) / (wz + 1e-8)
        gids = _nearest4(px, py)
        cnt = _counts(gids, lane_hw)
        N = cnt if N is None else N + cnt

    bx = wx * (1.0 / _GRID) - 0.5
    by = wy * (1.0 / _GRID) - 0.5
    x0 = jnp.floor(bx)
    y0 = jnp.floor(by)
    fx = bx - x0
    fy = by - y0
    x0c = jnp.clip(x0, 0.0, 31.0)
    x1c = jnp.clip(x0 + 1.0, 0.0, 31.0)
    y0c = jnp.clip(y0, 0.0, 31.0)
    y1c = jnp.clip(y0 + 1.0, 0.0, 31.0)
    A = ((lane_hw == y0c * 32.0 + x0c).astype(jnp.float32) * ((1 - fy) * (1 - fx))
         + (lane_hw == y0c * 32.0 + x1c).astype(jnp.float32) * ((1 - fy) * fx)
         + (lane_hw == y1c * 32.0 + x0c).astype(jnp.float32) * (fy * (1 - fx))
         + (lane_hw == y1c * 32.0 + x1c).astype(jnp.float32) * (fy * fx))
    wd = jax.lax.dot_general(A, Dt, (((1,), (0,)), ((), ())),
                             preferred_element_type=jnp.float32)
    wd = wd * jax.lax.rsqrt(jnp.sum(wd * wd, axis=1, keepdims=True) + 1e-12)

    pos = 2.0 - 2.0 * jnp.sum(kd * wd, axis=1, keepdims=True)

    S = jax.lax.dot_general(kd, D, (((1,), (0,)), ((), ())),
                            preferred_element_type=jnp.float32)
    X = 2.0 - 2.0 * S + 5.0 * N
    big = jnp.float32(1e30)
    fos_sum = jnp.float32(0.0)
    for _ in range(_NUM_NEG):
        m = jnp.min(X, axis=1, keepdims=True)
        sel = jnp.min(jnp.where(X == m, lane_hw, jnp.float32(hw)),
                      axis=1, keepdims=True)
        X = jnp.where(lane_hw == sel, big, X)
        t = jnp.maximum(pos - m + _MARGIN, 0.0)
        fos_sum = fos_sum + jnp.sum(t * t)

    nt = (((1,), (1,)), ((), ()))
    Km = jax.lax.dot_general(K, K, nt, preferred_element_type=jnp.float32)
    Wm = jax.lax.dot_general(N, W, nt, preferred_element_type=jnp.float32)
    kraw = 2.0 - 2.0 * jax.lax.dot_general(kd, kd, nt,
                                           preferred_element_type=jnp.float32)
    wraw = 2.0 - 2.0 * jax.lax.dot_general(wd, wd, nt,
                                           preferred_element_type=jnp.float32)
    Xa = kraw + 5.0 * Km
    Xb = wraw + 5.0 * Wm
    lane_n = jax.lax.broadcasted_iota(jnp.int32, (n, n), 1).astype(jnp.float32)
    sacc = jnp.zeros((n, 1), jnp.float32)
    for _ in range(_SOS_NEG):
        ma = jnp.min(Xa, axis=1, keepdims=True)
        sa = jnp.min(jnp.where(Xa == ma, lane_n, jnp.float32(n)),
                     axis=1, keepdims=True)
        va = jnp.sum(jnp.where(lane_n == sa, kraw, 0.0), axis=1, keepdims=True)
        Xa = jnp.where(lane_n == sa, big, Xa)
        mb = jnp.min(Xb, axis=1, keepdims=True)
        sb = jnp.min(jnp.where(Xb == mb, lane_n, jnp.float32(n)),
                     axis=1, keepdims=True)
        vb = jnp.sum(jnp.where(lane_n == sb, wraw, 0.0), axis=1, keepdims=True)
        Xb = jnp.where(lane_n == sb, big, Xb)
        d = va - vb
        sacc = sacc + d * d
    sos_sum = jnp.sum(jnp.sqrt(sacc + 1e-12))

    lane_o = jax.lax.broadcasted_iota(jnp.int32, (1, 128), 1)
    out_ref[0] = jnp.where(lane_o == 0, fos_sum,
                           jnp.where(lane_o == 1, sos_sum, 0.0))


@jax.jit
def kernel(kp1, w_kp1, kp1_desc, desc2, homo12):
    b, n, c = kp1_desc.shape
    h, w = desc2.shape[2], desc2.shape[3]
    hw = h * w
    D = desc2.reshape(b, c, hw)
    Dt = jnp.transpose(D, (0, 2, 1))
    homo_flat = homo12.reshape(b, 9)

    grid_spec = pltpu.PrefetchScalarGridSpec(
        num_scalar_prefetch=1,
        grid=(b,),
        in_specs=[
            pl.BlockSpec((1, n, 2), lambda i, s: (i, 0, 0)),
            pl.BlockSpec((1, n, 2), lambda i, s: (i, 0, 0)),
            pl.BlockSpec((1, n, c), lambda i, s: (i, 0, 0)),
            pl.BlockSpec((1, c, hw), lambda i, s: (i, 0, 0)),
            pl.BlockSpec((1, hw, c), lambda i, s: (i, 0, 0)),
        ],
        out_specs=pl.BlockSpec((1, 1, 128), lambda i, s: (i, 0, 0)),
    )
    part = pl.pallas_call(
        _loss_kernel,
        grid_spec=grid_spec,
        out_shape=jax.ShapeDtypeStruct((b, 1, 128), jnp.float32),
    )(homo_flat, kp1, w_kp1, kp1_desc, D, Dt)
    fos = jnp.sum(part[:, 0, 0]) / (b * n * _NUM_NEG)
    sos = jnp.sum(part[:, 0, 1]) / (b * n)
    return fos + sos
